# Initial kernel scaffold; baseline (speedup 1.0000x reference)
import jax
import jax.numpy as jnp
from jax.experimental import pallas as pl


def kernel(x, params):
    raise NotImplementedError("write your pallas kernel here")


# speedup vs baseline: 8.2120x; 8.2120x over previous
import math

import jax
import jax.numpy as jnp
import numpy as np
from jax import lax
from jax.experimental import pallas as pl
from jax.experimental.pallas import tpu as pltpu
from jax.experimental.pallas import tpu_sc as plsc


def _neg_dist_body(x_ref, xt_ref, o_ref):
    e = None
    sqi = None
    sqj = None
    for c in range(3):
        col = x_ref[0, :, c:c + 1]
        row = xt_ref[0, c:c + 1, :]
        colb = col.astype(jnp.bfloat16).astype(jnp.float32)
        rowb = row.astype(jnp.bfloat16).astype(jnp.float32)
        p = colb * rowb
        e = p if e is None else e + p
        si = col * col
        sj = row * row
        sqi = si if sqi is None else sqi + si
        sqj = sj if sqj is None else sqj + sj
    o_ref[0] = -((sqi + sqj) - 2.0 * e)


def _neg_dist(x, xt):
    B, N, F = x.shape
    return pl.pallas_call(
        _neg_dist_body,
        grid=(B,),
        in_specs=[
            pl.BlockSpec((1, N, F), lambda b: (b, 0, 0)),
            pl.BlockSpec((1, F, N), lambda b: (b, 0, 0)),
        ],
        out_specs=pl.BlockSpec((1, N, N), lambda b: (b, 0, 0)),
        out_shape=jax.ShapeDtypeStruct((B, N, N), jnp.float32),
    )(x, xt)


def _mm_many(triples):
    n = len(triples)

    def body(*refs):
        outs = refs[3 * n:]
        for i in range(n):
            h = refs[3 * i][...]
            w = refs[3 * i + 1][...]
            b = refs[3 * i + 2][...]
            outs[i][...] = jnp.dot(h, w, preferred_element_type=jnp.float32) + b

    out_shape = [
        jax.ShapeDtypeStruct((h.shape[0], w.shape[1]), jnp.float32)
        for (h, w, b) in triples
    ]
    flat = []
    for t in triples:
        flat.extend(t)
    return pl.pallas_call(body, out_shape=out_shape)(*flat)


def _gn_relu(h, gamma_rows, beta_rows):
    P = h.shape[0]
    G = P // 16
    inv = 1.0 / (16.0 * h.shape[1])
    gt_r = lax.broadcasted_iota(jnp.int32, (G, P), 0)
    gt_c = lax.broadcasted_iota(jnp.int32, (G, P), 1)
    Gt = (gt_c // 16 == gt_r).astype(jnp.float32)
    gm_r = lax.broadcasted_iota(jnp.int32, (P, G), 0)
    gm_c = lax.broadcasted_iota(jnp.int32, (P, G), 1)
    Gm = (gm_r // 16 == gm_c).astype(jnp.float32)
    s1 = jnp.sum(jnp.dot(Gt, h, preferred_element_type=jnp.float32),
                 axis=1, keepdims=True)
    s2 = jnp.sum(jnp.dot(Gt, h * h, preferred_element_type=jnp.float32),
                 axis=1, keepdims=True)
    m = s1 * inv
    v = s2 * inv - m * m
    sc = lax.rsqrt(v + 1e-5)
    mr = jnp.dot(Gm, m, preferred_element_type=jnp.float32)
    sr = jnp.dot(Gm, sc, preferred_element_type=jnp.float32)
    hn = (h - mr) * sr
    return jnp.maximum(hn * gamma_rows + beta_rows, 0.0)


def _fusion_body(h2_ref, wf2_ref, bf2_ref, f0_ref, f1_ref,
                 w1a_ref, w1b_ref, w1c_ref, b1_ref, gam1_ref, bet1_ref,
                 w2_ref, b2_ref, gam2_ref, bet2_ref, pe_ref, o_ref):
    f2 = jnp.dot(h2_ref[...], wf2_ref[...],
                 preferred_element_type=jnp.float32) + bf2_ref[...]
    h = (jnp.dot(f0_ref[...], w1a_ref[...], preferred_element_type=jnp.float32)
         + jnp.dot(f1_ref[...], w1b_ref[...], preferred_element_type=jnp.float32)
         + jnp.dot(f2, w1c_ref[...], preferred_element_type=jnp.float32)
         + b1_ref[...])
    h = _gn_relu(h, gam1_ref[...], bet1_ref[...])
    h = jnp.dot(h, w2_ref[...], preferred_element_type=jnp.float32) + b2_ref[...]
    h = _gn_relu(h, gam2_ref[...], bet2_ref[...])
    o_ref[...] = h + pe_ref[...]


def _segmax(idx_flat, stages):
    P = idx_flat.shape[0]
    NW = 32
    RP = P // NW
    ns = len(stages)
    shapes = [(int(k), int(g.shape[1])) for (g, a, k) in stages]
    mesh = plsc.VectorSubcoreMesh(core_axis_name="c", subcore_axis_name="s")

    scratch = [pltpu.VMEM((RP, idx_flat.shape[1]), jnp.int32)]
    for (k, C) in shapes:
        scratch += [
            pltpu.VMEM((k, C), jnp.float32),
            pltpu.VMEM((RP, C), jnp.float32),
            pltpu.VMEM((RP, C), jnp.float32),
        ]
    scratch.append(pltpu.SemaphoreType.DMA)

    out_type = [jax.ShapeDtypeStruct((P, C), jnp.float32) for (k, C) in shapes]

    def body(*refs):
        idx_hbm = refs[0]
        gas = refs[1:1 + 2 * ns]
        outs = refs[1 + 2 * ns:1 + 3 * ns]
        idx_v = refs[1 + 3 * ns]
        stage_scr = refs[2 + 3 * ns:2 + 6 * ns]
        sem = refs[-1]
        wid = lax.axis_index("s") * 2 + lax.axis_index("c")
        base = pl.multiple_of(wid * RP, RP)
        pltpu.sync_copy(idx_hbm.at[pl.ds(base, RP)], idx_v)
        for si in range(ns):
            k, C = shapes[si]
            g_hbm, a_hbm = gas[2 * si], gas[2 * si + 1]
            rows_v, a_v, out_v = stage_scr[3 * si:3 * si + 3]
            o_hbm = outs[si]
            nch = C // 16
            CH = min(nch, 16)
            ngrp = nch // CH
            pltpu.sync_copy(a_hbm.at[pl.ds(base, RP)], a_v)

            def point(p, _, rows_v=rows_v, a_v=a_v, out_v=out_v,
                      g_hbm=g_hbm, k=k, CH=CH, ngrp=ngrp):
                pltpu.async_copy(
                    g_hbm.at[idx_v.at[p, pl.ds(0, k)]], rows_v, sem).wait()
                for grp in range(ngrp):
                    off0 = grp * CH * 16

                    def red(j, accs, rows_v=rows_v, off0=off0, CH=CH):
                        return tuple(
                            jnp.maximum(accs[t],
                                        rows_v[j, pl.ds(off0 + 16 * t, 16)])
                            for t in range(CH))

                    accs = tuple(rows_v[0, pl.ds(off0 + 16 * t, 16)]
                                 for t in range(CH))
                    accs = lax.fori_loop(1, k, red, accs)
                    for t in range(CH):
                        sl = pl.ds(off0 + 16 * t, 16)
                        out_v[p, sl] = jnp.maximum(accs[t] + a_v[p, sl], 0.0)
                return 0

            lax.fori_loop(0, RP, point, 0)
            pltpu.sync_copy(out_v, o_hbm.at[pl.ds(base, RP)])

    flat = [idx_flat]
    for (g, a, k) in stages:
        flat += [g, a]
    res = pl.kernel(body, out_type=out_type, mesh=mesh,
                    scratch_types=scratch)(*flat)
    return list(res) if ns > 1 else [res[0] if isinstance(res, (tuple, list)) else res]


def _pos_encoding(N, d):
    position = jnp.arange(N, dtype=jnp.float32)[:, None]
    div = jnp.exp(jnp.arange(0, d, 2, dtype=jnp.float32)
                  * (-math.log(10000.0) / d))
    pe = jnp.zeros((N, d), jnp.float32)
    pe = pe.at[:, 0::2].set(jnp.sin(position * div))
    pe = pe.at[:, 1::2].set(jnp.cos(position * div))
    return pe


def _split_w(W):
    C = W.shape[0] // 2
    return W[:C] - W[C:], W[C:]


def kernel(x, params):
    B, N, F = x.shape
    P = B * N

    xt = jnp.transpose(x, (0, 2, 1))
    negd = _neg_dist(x, xt)
    _, idx = lax.top_k(negd, 64)
    idx_flat = (idx.astype(jnp.int32)
                + (jnp.arange(B, dtype=jnp.int32) * N)[:, None, None]
                ).reshape(P, 64)

    xf = x.reshape(P, F)
    xp = jnp.pad(xf, ((0, 0), (0, 2)))

    enc = [params["enc0"], params["enc1"], params["enc2"]]
    ks = (16, 32, 64)

    tri = []
    for e in enc:
        W, b = e["layers"][0]
        wa, wg = _split_w(W)
        wa = jnp.pad(wa, ((0, 2), (0, 0)))
        wg = jnp.pad(wg, ((0, 2), (0, 0)))
        tri += [(xp, wa, b[None, :]), (xp, wg, jnp.zeros((1, wg.shape[1]), jnp.float32))]
    a0, g0, a1, g1, a2, g2 = _mm_many(tri)
    h0, h1, h2 = _segmax(idx_flat, [(g0, a0, 16), (g1, a1, 32), (g2, a2, 64)])

    tri = []
    for e, h in zip(enc, (h0, h1, h2)):
        W, b = e["layers"][1]
        wa, wg = _split_w(W)
        tri += [(h, wa, b[None, :]), (h, wg, jnp.zeros((1, wg.shape[1]), jnp.float32))]
    a0, g0, a1, g1, a2, g2 = _mm_many(tri)
    h0, h1, h2 = _segmax(idx_flat, [(g0, a0, 16), (g1, a1, 32), (g2, a2, 64)])

    W2c, b2c = enc[2]["layers"][2]
    wa2, wg2 = _split_w(W2c)
    tri = [
        (h0, enc[0]["final"][0], enc[0]["final"][1][None, :]),
        (h1, enc[1]["final"][0], enc[1]["final"][1][None, :]),
        (h2, wa2, b2c[None, :]),
        (h2, wg2, jnp.zeros((1, wg2.shape[1]), jnp.float32)),
    ]
    f0, f1, a2, g2 = _mm_many(tri)
    (h2,) = _segmax(idx_flat, [(g2, a2, 64)])

    f = params["fusion"]
    W1 = f["W1"]
    pe = _pos_encoding(N, 512)
    pe2 = jnp.concatenate([pe, pe], axis=0)
    gam1 = jnp.concatenate([f["g1"], f["g1"]])[:, None]
    bet1 = jnp.concatenate([f["be1"], f["be1"]])[:, None]
    gam2 = jnp.concatenate([f["g2"], f["g2"]])[:, None]
    bet2 = jnp.concatenate([f["be2"], f["be2"]])[:, None]

    out = pl.pallas_call(
        _fusion_body,
        out_shape=jax.ShapeDtypeStruct((P, 512), jnp.float32),
    )(h2, enc[2]["final"][0], enc[2]["final"][1][None, :], f0, f1,
      W1[0:256], W1[256:768], W1[768:1280], f["b1"][None, :], gam1, bet1,
      f["W2"], f["b2"][None, :], gam2, bet2, pe2)

    return out.reshape(B, N, 512)


# speedup vs baseline: 10.6647x; 1.2987x over previous
import math

import jax
import jax.numpy as jnp
import numpy as np
from jax import lax
from jax.experimental import pallas as pl
from jax.experimental.pallas import tpu as pltpu
from jax.experimental.pallas import tpu_sc as plsc


def _neg_dist_body(x_ref, xt_ref, o_ref):
    e = None
    sqi = None
    sqj = None
    for c in range(3):
        col = x_ref[0, :, c:c + 1]
        row = xt_ref[0, c:c + 1, :]
        colb = col.astype(jnp.bfloat16).astype(jnp.float32)
        rowb = row.astype(jnp.bfloat16).astype(jnp.float32)
        p = colb * rowb
        e = p if e is None else e + p
        si = col * col
        sj = row * row
        sqi = si if sqi is None else sqi + si
        sqj = sj if sqj is None else sqj + sj
    o_ref[0] = -((sqi + sqj) - 2.0 * e)


def _neg_dist(x, xt):
    B, N, F = x.shape
    return pl.pallas_call(
        _neg_dist_body,
        grid=(B,),
        in_specs=[
            pl.BlockSpec((1, N, F), lambda b: (b, 0, 0)),
            pl.BlockSpec((1, F, N), lambda b: (b, 0, 0)),
        ],
        out_specs=pl.BlockSpec((1, N, N), lambda b: (b, 0, 0)),
        out_shape=jax.ShapeDtypeStruct((B, N, N), jnp.float32),
    )(x, xt)


def _tc_round(entries):
    n_in = {"x_ag": 4, "am_ag": 5, "am_f": 4}
    flat, out_shape, modes = [], [], []
    for e in entries:
        modes.append(e[0])
        flat.extend(e[1:])
        P = e[1].shape[0]
        if e[0] == "x_ag":
            out_shape += [jax.ShapeDtypeStruct((P, e[2].shape[1]), jnp.float32)] * 2
        elif e[0] == "am_ag":
            out_shape += [jax.ShapeDtypeStruct((P, e[3].shape[1]), jnp.float32)] * 2
        else:
            out_shape += [jax.ShapeDtypeStruct((P, e[3].shape[1]), jnp.float32)]

    n_flat = len(flat)

    def body(*refs):
        outs = refs[n_flat:]
        ri = 0
        oi = 0
        for mode in modes:
            if mode == "x_ag":
                h = refs[ri][...]
                wa, wg, b = refs[ri + 1][...], refs[ri + 2][...], refs[ri + 3][...]
            elif mode == "am_ag":
                h = jnp.maximum(refs[ri][...] + refs[ri + 1][...], 0.0)
                wa, wg, b = refs[ri + 2][...], refs[ri + 3][...], refs[ri + 4][...]
            else:
                h = jnp.maximum(refs[ri][...] + refs[ri + 1][...], 0.0)
                w, b = refs[ri + 2][...], refs[ri + 3][...]
                outs[oi][...] = jnp.dot(h, w, preferred_element_type=jnp.float32) + b
                ri += 4
                oi += 1
                continue
            outs[oi][...] = jnp.dot(h, wa, preferred_element_type=jnp.float32) + b
            outs[oi + 1][...] = jnp.dot(h, wg, preferred_element_type=jnp.float32)
            ri += n_in[mode]
            oi += 2

    return pl.pallas_call(body, out_shape=out_shape)(*flat)


def _gn_relu(h, gamma_rows, beta_rows):
    P = h.shape[0]
    G = P // 16
    inv = 1.0 / (16.0 * h.shape[1])
    gt_r = lax.broadcasted_iota(jnp.int32, (G, P), 0)
    gt_c = lax.broadcasted_iota(jnp.int32, (G, P), 1)
    Gt = (gt_c // 16 == gt_r).astype(jnp.float32)
    gm_r = lax.broadcasted_iota(jnp.int32, (P, G), 0)
    gm_c = lax.broadcasted_iota(jnp.int32, (P, G), 1)
    Gm = (gm_r // 16 == gm_c).astype(jnp.float32)
    s1 = jnp.sum(jnp.dot(Gt, h, preferred_element_type=jnp.float32),
                 axis=1, keepdims=True)
    s2 = jnp.sum(jnp.dot(Gt, h * h, preferred_element_type=jnp.float32),
                 axis=1, keepdims=True)
    m = s1 * inv
    v = s2 * inv - m * m
    sc = lax.rsqrt(v + 1e-5)
    mr = jnp.dot(Gm, m, preferred_element_type=jnp.float32)
    sr = jnp.dot(Gm, sc, preferred_element_type=jnp.float32)
    hn = (h - mr) * sr
    return jnp.maximum(hn * gamma_rows + beta_rows, 0.0)


def _fusion_body(a2_ref, m2_ref, wf2_ref, bf2_ref, f0_ref, f1_ref,
                 w1a_ref, w1b_ref, w1c_ref, b1_ref, gam1_ref, bet1_ref,
                 w2_ref, b2_ref, gam2_ref, bet2_ref, pe_ref, o_ref):
    h2 = jnp.maximum(a2_ref[...] + m2_ref[...], 0.0)
    f2 = jnp.dot(h2, wf2_ref[...],
                 preferred_element_type=jnp.float32) + bf2_ref[...]
    h = (jnp.dot(f0_ref[...], w1a_ref[...], preferred_element_type=jnp.float32)
         + jnp.dot(f1_ref[...], w1b_ref[...], preferred_element_type=jnp.float32)
         + jnp.dot(f2, w1c_ref[...], preferred_element_type=jnp.float32)
         + b1_ref[...])
    h = _gn_relu(h, gam1_ref[...], bet1_ref[...])
    h = jnp.dot(h, w2_ref[...], preferred_element_type=jnp.float32) + b2_ref[...]
    h = _gn_relu(h, gam2_ref[...], bet2_ref[...])
    o_ref[...] = h + pe_ref[...]


def _segmax(stages):
    P = stages[0][1].shape[0]
    NW = 32
    RP = P // NW
    ns = len(stages)
    plan = []
    for (idxf, g, k) in stages:
        k = int(k)
        C = int(g.shape[1])
        PB = max(1, min(128 // k, 8192 // (k * C), RP))
        plan.append((k, C, PB))
    mesh = plsc.VectorSubcoreMesh(core_axis_name="c", subcore_axis_name="s")

    scratch = []
    for (k, C, PB) in plan:
        scratch += [
            pltpu.VMEM((RP * k,), jnp.int32),
            pltpu.VMEM((PB * k, C), jnp.float32),
            pltpu.VMEM((PB * k, C), jnp.float32),
            pltpu.VMEM((RP, C), jnp.float32),
        ]
    scratch.append(pltpu.SemaphoreType.DMA)

    out_type = [jax.ShapeDtypeStruct((P, C), jnp.float32) for (k, C, PB) in plan]

    def body(*refs):
        outs = refs[2 * ns:3 * ns]
        sem = refs[-1]
        wid = lax.axis_index("s") * 2 + lax.axis_index("c")
        base = pl.multiple_of(wid * RP, RP)
        for si in range(ns):
            k, C, PB = plan[si]
            idxf_hbm, g_hbm = refs[2 * si], refs[2 * si + 1]
            o_hbm = outs[si]
            idx_v, rows0, rows1, out_v = refs[3 * ns + 4 * si:3 * ns + 4 * si + 4]
            W = PB * k
            NG = RP // PB
            nch = C // 16
            CH = min(nch, 16)
            ngrp = nch // CH
            pltpu.sync_copy(idxf_hbm.at[pl.ds(base * k, RP * k)], idx_v)

            def start(gi, buf, idx_v=idx_v, g_hbm=g_hbm, W=W, sem=sem):
                off = pl.multiple_of(gi * W, 8)
                pltpu.async_copy(g_hbm.at[idx_v.at[pl.ds(off, W)]], buf, sem)

            def drain(buf, g_hbm=g_hbm, W=W, sem=sem):
                pltpu.make_async_copy(g_hbm.at[pl.ds(0, W)], buf, sem).wait()

            def reduce(gi, buf, out_v=out_v, k=k, PB=PB, CH=CH, ngrp=ngrp):
                for pi in range(PB):
                    r0 = pi * k
                    for grp in range(ngrp):
                        off0 = grp * CH * 16

                        def red(j, accs, buf=buf, r0=r0, off0=off0, CH=CH):
                            return tuple(
                                jnp.maximum(accs[t],
                                            buf[r0 + j, pl.ds(off0 + 16 * t, 16)])
                                for t in range(CH))

                        accs = tuple(buf[r0, pl.ds(off0 + 16 * t, 16)]
                                     for t in range(CH))
                        accs = lax.fori_loop(1, k, red, accs)
                        prow = gi * PB + pi
                        for t in range(CH):
                            out_v[prow, pl.ds(off0 + 16 * t, 16)] = accs[t]

            start(0, rows0)

            def pair(pq, _, rows0=rows0, rows1=rows1, NG=NG,
                     start=start, drain=drain, reduce=reduce):
                g0 = 2 * pq
                drain(rows0)
                start(g0 + 1, rows1)
                reduce(g0, rows0)
                drain(rows1)

                @pl.when(pq < NG // 2 - 1)
                def _():
                    start(g0 + 2, rows0)

                reduce(g0 + 1, rows1)
                return 0

            lax.fori_loop(0, NG // 2, pair, 0)
            pltpu.sync_copy(out_v, o_hbm.at[pl.ds(base, RP)])

    flat = []
    for (idxf, g, k) in stages:
        flat += [idxf, g]
    res = pl.kernel(body, out_type=out_type, mesh=mesh,
                    scratch_types=scratch)(*flat)
    if ns == 1:
        return [res[0] if isinstance(res, (tuple, list)) else res]
    return list(res)


def _pos_encoding(N, d):
    position = jnp.arange(N, dtype=jnp.float32)[:, None]
    div = jnp.exp(jnp.arange(0, d, 2, dtype=jnp.float32)
                  * (-math.log(10000.0) / d))
    pe = jnp.zeros((N, d), jnp.float32)
    pe = pe.at[:, 0::2].set(jnp.sin(position * div))
    pe = pe.at[:, 1::2].set(jnp.cos(position * div))
    return pe


def _split_w(W):
    C = W.shape[0] // 2
    return W[:C] - W[C:], W[C:]


def kernel(x, params):
    B, N, F = x.shape
    P = B * N

    xt = jnp.transpose(x, (0, 2, 1))
    negd = _neg_dist(x, xt)
    _, idx = lax.top_k(negd, 64)
    idx_flat = (idx.astype(jnp.int32)
                + (jnp.arange(B, dtype=jnp.int32) * N)[:, None, None]
                ).reshape(P, 64)
    idxf = {k: idx_flat[:, :k].reshape(-1) for k in (16, 32, 64)}

    xf = x.reshape(P, F)
    xp = jnp.pad(xf, ((0, 0), (0, 2)))

    enc = [params["enc0"], params["enc1"], params["enc2"]]
    ks = (16, 32, 64)

    entries = []
    for e in enc:
        W, b = e["layers"][0]
        wa, wg = _split_w(W)
        wa = jnp.pad(wa, ((0, 2), (0, 0)))
        wg = jnp.pad(wg, ((0, 2), (0, 0)))
        entries.append(("x_ag", xp, wa, wg, b[None, :]))
    a0, g0, a1, g1, a2, g2 = _tc_round(entries)
    m0, m1, m2 = _segmax([(idxf[16], g0, 16), (idxf[32], g1, 32), (idxf[64], g2, 64)])

    entries = []
    for e, a, m in zip(enc, (a0, a1, a2), (m0, m1, m2)):
        W, b = e["layers"][1]
        wa, wg = _split_w(W)
        entries.append(("am_ag", a, m, wa, wg, b[None, :]))
    a0, g0, a1, g1, a2, g2 = _tc_round(entries)
    m0, m1, m2 = _segmax([(idxf[16], g0, 16), (idxf[32], g1, 32), (idxf[64], g2, 64)])

    W2c, b2c = enc[2]["layers"][2]
    wa2, wg2 = _split_w(W2c)
    f0, f1, a2, g2 = _tc_round([
        ("am_f", a0, m0, enc[0]["final"][0], enc[0]["final"][1][None, :]),
        ("am_f", a1, m1, enc[1]["final"][0], enc[1]["final"][1][None, :]),
        ("am_ag", a2, m2, wa2, wg2, b2c[None, :]),
    ])
    (m2,) = _segmax([(idxf[64], g2, 64)])

    f = params["fusion"]
    W1 = f["W1"]
    pe = _pos_encoding(N, 512)
    pe2 = jnp.concatenate([pe, pe], axis=0)
    gam1 = jnp.concatenate([f["g1"], f["g1"]])[:, None]
    bet1 = jnp.concatenate([f["be1"], f["be1"]])[:, None]
    gam2 = jnp.concatenate([f["g2"], f["g2"]])[:, None]
    bet2 = jnp.concatenate([f["be2"], f["be2"]])[:, None]

    out = pl.pallas_call(
        _fusion_body,
        out_shape=jax.ShapeDtypeStruct((P, 512), jnp.float32),
    )(a2, m2, enc[2]["final"][0], enc[2]["final"][1][None, :], f0, f1,
      W1[0:256], W1[256:768], W1[768:1280], f["b1"][None, :], gam1, bet1,
      f["W2"], f["b2"][None, :], gam2, bet2, pe2)

    return out.reshape(B, N, 512)


# speedup vs baseline: 11.2016x; 1.0503x over previous
import math

import jax
import jax.numpy as jnp
import numpy as np
from jax import lax
from jax.experimental import pallas as pl
from jax.experimental.pallas import tpu as pltpu
from jax.experimental.pallas import tpu_sc as plsc


def _neg_dist_body(x_ref, xt_ref, o_ref):
    e = None
    sqi = None
    sqj = None
    for c in range(3):
        col = x_ref[0, :, c:c + 1]
        row = xt_ref[0, c:c + 1, :]
        colb = col.astype(jnp.bfloat16).astype(jnp.float32)
        rowb = row.astype(jnp.bfloat16).astype(jnp.float32)
        p = colb * rowb
        e = p if e is None else e + p
        si = col * col
        sj = row * row
        sqi = si if sqi is None else sqi + si
        sqj = sj if sqj is None else sqj + sj
    o_ref[0] = -((sqi + sqj) - 2.0 * e)


def _neg_dist(x, xt):
    B, N, F = x.shape
    return pl.pallas_call(
        _neg_dist_body,
        grid=(B,),
        in_specs=[
            pl.BlockSpec((1, N, F), lambda b: (b, 0, 0)),
            pl.BlockSpec((1, F, N), lambda b: (b, 0, 0)),
        ],
        out_specs=pl.BlockSpec((1, N, N), lambda b: (b, 0, 0)),
        out_shape=jax.ShapeDtypeStruct((B, N, N), jnp.float32),
    )(x, xt)


def _tc_round(entries):
    n_in = {"x_ag": 4, "am_ag": 5, "am_f": 4}
    flat, out_shape, modes = [], [], []
    for e in entries:
        modes.append(e[0])
        flat.extend(e[1:])
        P = e[1].shape[0]
        if e[0] == "x_ag":
            out_shape += [jax.ShapeDtypeStruct((P, e[2].shape[1]), jnp.float32)] * 2
        elif e[0] == "am_ag":
            out_shape += [jax.ShapeDtypeStruct((P, e[3].shape[1]), jnp.float32)] * 2
        else:
            out_shape += [jax.ShapeDtypeStruct((P, e[3].shape[1]), jnp.float32)]

    n_flat = len(flat)

    def body(*refs):
        outs = refs[n_flat:]
        ri = 0
        oi = 0
        for mode in modes:
            if mode == "x_ag":
                h = refs[ri][...]
                wa, wg, b = refs[ri + 1][...], refs[ri + 2][...], refs[ri + 3][...]
            elif mode == "am_ag":
                h = jnp.maximum(refs[ri][...] + refs[ri + 1][...], 0.0)
                wa, wg, b = refs[ri + 2][...], refs[ri + 3][...], refs[ri + 4][...]
            else:
                h = jnp.maximum(refs[ri][...] + refs[ri + 1][...], 0.0)
                w, b = refs[ri + 2][...], refs[ri + 3][...]
                outs[oi][...] = jnp.dot(h, w, preferred_element_type=jnp.float32) + b
                ri += 4
                oi += 1
                continue
            outs[oi][...] = jnp.dot(h, wa, preferred_element_type=jnp.float32) + b
            outs[oi + 1][...] = jnp.dot(h, wg, preferred_element_type=jnp.float32)
            ri += n_in[mode]
            oi += 2

    return pl.pallas_call(body, out_shape=out_shape)(*flat)


def _gn_relu(h, gamma_rows, beta_rows):
    P = h.shape[0]
    G = P // 16
    inv = 1.0 / (16.0 * h.shape[1])
    gt_r = lax.broadcasted_iota(jnp.int32, (G, P), 0)
    gt_c = lax.broadcasted_iota(jnp.int32, (G, P), 1)
    Gt = (gt_c // 16 == gt_r).astype(jnp.float32)
    gm_r = lax.broadcasted_iota(jnp.int32, (P, G), 0)
    gm_c = lax.broadcasted_iota(jnp.int32, (P, G), 1)
    Gm = (gm_r // 16 == gm_c).astype(jnp.float32)
    s1 = jnp.sum(jnp.dot(Gt, h, preferred_element_type=jnp.float32),
                 axis=1, keepdims=True)
    s2 = jnp.sum(jnp.dot(Gt, h * h, preferred_element_type=jnp.float32),
                 axis=1, keepdims=True)
    m = s1 * inv
    v = s2 * inv - m * m
    sc = lax.rsqrt(v + 1e-5)
    mr = jnp.dot(Gm, m, preferred_element_type=jnp.float32)
    sr = jnp.dot(Gm, sc, preferred_element_type=jnp.float32)
    hn = (h - mr) * sr
    return jnp.maximum(hn * gamma_rows + beta_rows, 0.0)


def _fusion_body(a2_ref, m2_ref, wf2_ref, bf2_ref, f0_ref, f1_ref,
                 w1a_ref, w1b_ref, w1c_ref, b1_ref, gam1_ref, bet1_ref,
                 w2_ref, b2_ref, gam2_ref, bet2_ref, pe_ref, o_ref):
    h2 = jnp.maximum(a2_ref[...] + m2_ref[...], 0.0)
    f2 = jnp.dot(h2, wf2_ref[...],
                 preferred_element_type=jnp.float32) + bf2_ref[...]
    h = (jnp.dot(f0_ref[...], w1a_ref[...], preferred_element_type=jnp.float32)
         + jnp.dot(f1_ref[...], w1b_ref[...], preferred_element_type=jnp.float32)
         + jnp.dot(f2, w1c_ref[...], preferred_element_type=jnp.float32)
         + b1_ref[...])
    h = _gn_relu(h, gam1_ref[...], bet1_ref[...])
    h = jnp.dot(h, w2_ref[...], preferred_element_type=jnp.float32) + b2_ref[...]
    h = _gn_relu(h, gam2_ref[...], bet2_ref[...])
    o_ref[...] = h + pe_ref[...]


def _segmax(stages):
    P = stages[0][1].shape[0]
    NW = 32
    RP = P // NW
    ns = len(stages)

    def make_plan(cap):
        plan = []
        words = 0
        for (idxf, g, k) in stages:
            k = int(k)
            C = int(g.shape[1])
            PB = max(1, min(128 // k, cap // (k * C), RP))
            plan.append((k, C, PB))
            words += RP * k + 2 * PB * k * C + RP * C
        return plan, words

    plan, words = make_plan(16384)
    if words > 126000:
        plan, words = make_plan(8192)
    mesh = plsc.VectorSubcoreMesh(core_axis_name="c", subcore_axis_name="s")

    scratch = []
    for (k, C, PB) in plan:
        scratch += [
            pltpu.VMEM((RP * k,), jnp.int32),
            pltpu.VMEM((PB * k, C), jnp.float32),
            pltpu.VMEM((PB * k, C), jnp.float32),
            pltpu.VMEM((RP, C), jnp.float32),
        ]
    scratch.append(pltpu.SemaphoreType.DMA)
    scratch.append(pltpu.SemaphoreType.DMA)

    out_type = [jax.ShapeDtypeStruct((P, C), jnp.float32) for (k, C, PB) in plan]

    def body(*refs):
        outs = refs[2 * ns:3 * ns]
        sem = refs[-2]
        osem = refs[-1]
        wid = lax.axis_index("s") * 2 + lax.axis_index("c")
        base = pl.multiple_of(wid * RP, RP)
        for si in range(ns):
            k, C, PB = plan[si]
            idx_v = refs[3 * ns + 4 * si]
            pltpu.sync_copy(refs[2 * si].at[pl.ds(base * k, RP * k)], idx_v)
        for si in range(ns):
            k, C, PB = plan[si]
            g_hbm = refs[2 * si + 1]
            o_hbm = outs[si]
            idx_v, rows0, rows1, out_v = refs[3 * ns + 4 * si:3 * ns + 4 * si + 4]
            W = PB * k
            NG = RP // PB
            nch = C // 16
            CH = min(nch, 16)
            ngrp = nch // CH

            def start(gi, buf, idx_v=idx_v, g_hbm=g_hbm, W=W, sem=sem):
                off = pl.multiple_of(gi * W, 8)
                pltpu.async_copy(g_hbm.at[idx_v.at[pl.ds(off, W)]], buf, sem)

            def drain(buf, g_hbm=g_hbm, W=W, sem=sem):
                pltpu.make_async_copy(g_hbm.at[pl.ds(0, W)], buf, sem).wait()

            def reduce(gi, buf, out_v=out_v, k=k, PB=PB, CH=CH, ngrp=ngrp):
                for pi in range(PB):
                    r0 = pi * k
                    for grp in range(ngrp):
                        off0 = grp * CH * 16

                        def red(blk, accs, buf=buf, r0=r0, off0=off0, CH=CH):
                            j = 1 + 3 * blk
                            for u in range(3):
                                accs = tuple(
                                    jnp.maximum(
                                        accs[t],
                                        buf[r0 + j + u, pl.ds(off0 + 16 * t, 16)])
                                    for t in range(CH))
                            return accs

                        accs = tuple(buf[r0, pl.ds(off0 + 16 * t, 16)]
                                     for t in range(CH))
                        accs = lax.fori_loop(0, (k - 1) // 3, red, accs)
                        for r in range(1 + 3 * ((k - 1) // 3), k):
                            accs = tuple(
                                jnp.maximum(accs[t],
                                            buf[r0 + r, pl.ds(off0 + 16 * t, 16)])
                                for t in range(CH))
                        prow = gi * PB + pi
                        for t in range(CH):
                            out_v[prow, pl.ds(off0 + 16 * t, 16)] = accs[t]

            start(0, rows0)

            def pair(pq, _, rows0=rows0, rows1=rows1, NG=NG,
                     start=start, drain=drain, reduce=reduce):
                g0 = 2 * pq
                drain(rows0)
                start(g0 + 1, rows1)
                reduce(g0, rows0)
                drain(rows1)

                @pl.when(pq < NG // 2 - 1)
                def _():
                    start(g0 + 2, rows0)

                reduce(g0 + 1, rows1)
                return 0

            lax.fori_loop(0, NG // 2, pair, 0)
            pltpu.async_copy(out_v, o_hbm.at[pl.ds(base, RP)], osem)
        for si in range(ns):
            k, C, PB = plan[si]
            out_v = refs[3 * ns + 4 * si + 3]
            pltpu.make_async_copy(outs[si].at[pl.ds(base, RP)], out_v, osem).wait()

    flat = []
    for (idxf, g, k) in stages:
        flat += [idxf, g]
    res = pl.kernel(body, out_type=out_type, mesh=mesh,
                    scratch_types=scratch)(*flat)
    if ns == 1:
        return [res[0] if isinstance(res, (tuple, list)) else res]
    return list(res)


def _pos_encoding(N, d):
    position = jnp.arange(N, dtype=jnp.float32)[:, None]
    div = jnp.exp(jnp.arange(0, d, 2, dtype=jnp.float32)
                  * (-math.log(10000.0) / d))
    pe = jnp.zeros((N, d), jnp.float32)
    pe = pe.at[:, 0::2].set(jnp.sin(position * div))
    pe = pe.at[:, 1::2].set(jnp.cos(position * div))
    return pe


def _split_w(W):
    C = W.shape[0] // 2
    return W[:C] - W[C:], W[C:]


def kernel(x, params):
    B, N, F = x.shape
    P = B * N

    xt = jnp.transpose(x, (0, 2, 1))
    negd = _neg_dist(x, xt)
    _, idx = lax.top_k(negd, 64)
    idx_flat = (idx.astype(jnp.int32)
                + (jnp.arange(B, dtype=jnp.int32) * N)[:, None, None]
                ).reshape(P, 64)
    idxf = {k: idx_flat[:, :k].reshape(-1) for k in (16, 32, 64)}

    xf = x.reshape(P, F)
    xp = jnp.pad(xf, ((0, 0), (0, 2)))

    enc = [params["enc0"], params["enc1"], params["enc2"]]
    ks = (16, 32, 64)

    entries = []
    for e in enc:
        W, b = e["layers"][0]
        wa, wg = _split_w(W)
        wa = jnp.pad(wa, ((0, 2), (0, 0)))
        wg = jnp.pad(wg, ((0, 2), (0, 0)))
        entries.append(("x_ag", xp, wa, wg, b[None, :]))
    a0, g0, a1, g1, a2, g2 = _tc_round(entries)
    m0, m1, m2 = _segmax([(idxf[16], g0, 16), (idxf[32], g1, 32), (idxf[64], g2, 64)])

    entries = []
    for e, a, m in zip(enc, (a0, a1, a2), (m0, m1, m2)):
        W, b = e["layers"][1]
        wa, wg = _split_w(W)
        entries.append(("am_ag", a, m, wa, wg, b[None, :]))
    a0, g0, a1, g1, a2, g2 = _tc_round(entries)
    m0, m1, m2 = _segmax([(idxf[16], g0, 16), (idxf[32], g1, 32), (idxf[64], g2, 64)])

    W2c, b2c = enc[2]["layers"][2]
    wa2, wg2 = _split_w(W2c)
    f0, f1, a2, g2 = _tc_round([
        ("am_f", a0, m0, enc[0]["final"][0], enc[0]["final"][1][None, :]),
        ("am_f", a1, m1, enc[1]["final"][0], enc[1]["final"][1][None, :]),
        ("am_ag", a2, m2, wa2, wg2, b2c[None, :]),
    ])
    (m2,) = _segmax([(idxf[64], g2, 64)])

    f = params["fusion"]
    W1 = f["W1"]
    pe = _pos_encoding(N, 512)
    pe2 = jnp.concatenate([pe, pe], axis=0)
    gam1 = jnp.concatenate([f["g1"], f["g1"]])[:, None]
    bet1 = jnp.concatenate([f["be1"], f["be1"]])[:, None]
    gam2 = jnp.concatenate([f["g2"], f["g2"]])[:, None]
    bet2 = jnp.concatenate([f["be2"], f["be2"]])[:, None]

    out = pl.pallas_call(
        _fusion_body,
        out_shape=jax.ShapeDtypeStruct((P, 512), jnp.float32),
    )(a2, m2, enc[2]["final"][0], enc[2]["final"][1][None, :], f0, f1,
      W1[0:256], W1[256:768], W1[768:1280], f["b1"][None, :], gam1, bet1,
      f["W2"], f["b2"][None, :], gam2, bet2, pe2)

    return out.reshape(B, N, 512)


# speedup vs baseline: 11.4143x; 1.0190x over previous
import math

import jax
import jax.numpy as jnp
import numpy as np
from jax import lax
from jax.experimental import pallas as pl
from jax.experimental.pallas import tpu as pltpu
from jax.experimental.pallas import tpu_sc as plsc


def _neg_dist_body(x_ref, xt_ref, o_ref):
    e = None
    sqi = None
    sqj = None
    for c in range(3):
        col = x_ref[0, :, c:c + 1]
        row = xt_ref[0, c:c + 1, :]
        colb = col.astype(jnp.bfloat16).astype(jnp.float32)
        rowb = row.astype(jnp.bfloat16).astype(jnp.float32)
        p = colb * rowb
        e = p if e is None else e + p
        si = col * col
        sj = row * row
        sqi = si if sqi is None else sqi + si
        sqj = sj if sqj is None else sqj + sj
    o_ref[0] = -((sqi + sqj) - 2.0 * e)


def _neg_dist(x, xt):
    B, N, F = x.shape
    return pl.pallas_call(
        _neg_dist_body,
        grid=(B,),
        in_specs=[
            pl.BlockSpec((1, N, F), lambda b: (b, 0, 0)),
            pl.BlockSpec((1, F, N), lambda b: (b, 0, 0)),
        ],
        out_specs=pl.BlockSpec((1, N, N), lambda b: (b, 0, 0)),
        out_shape=jax.ShapeDtypeStruct((B, N, N), jnp.float32),
    )(x, xt)


def _tc_round(entries):
    n_in = {"x_ag": 4, "am_ag": 5, "am_f": 4}
    flat, out_shape, modes = [], [], []
    for e in entries:
        modes.append(e[0])
        flat.extend(e[1:])
        P = e[1].shape[0]
        if e[0] == "x_ag":
            out_shape += [jax.ShapeDtypeStruct((P, e[2].shape[1]), jnp.float32)] * 2
        elif e[0] == "am_ag":
            out_shape += [jax.ShapeDtypeStruct((P, e[3].shape[1]), jnp.float32)] * 2
        else:
            out_shape += [jax.ShapeDtypeStruct((P, e[3].shape[1]), jnp.float32)]

    n_flat = len(flat)

    def body(*refs):
        outs = refs[n_flat:]
        ri = 0
        oi = 0
        for mode in modes:
            if mode == "x_ag":
                h = refs[ri][...]
                wa, wg, b = refs[ri + 1][...], refs[ri + 2][...], refs[ri + 3][...]
            elif mode == "am_ag":
                h = jnp.maximum(refs[ri][...] + refs[ri + 1][...], 0.0)
                wa, wg, b = refs[ri + 2][...], refs[ri + 3][...], refs[ri + 4][...]
            else:
                h = jnp.maximum(refs[ri][...] + refs[ri + 1][...], 0.0)
                w, b = refs[ri + 2][...], refs[ri + 3][...]
                outs[oi][...] = jnp.dot(h, w, preferred_element_type=jnp.float32) + b
                ri += 4
                oi += 1
                continue
            outs[oi][...] = jnp.dot(h, wa, preferred_element_type=jnp.float32) + b
            outs[oi + 1][...] = jnp.dot(h, wg, preferred_element_type=jnp.float32)
            ri += n_in[mode]
            oi += 2

    return pl.pallas_call(body, out_shape=out_shape)(*flat)


def _gn_relu(h, gamma_rows, beta_rows):
    P = h.shape[0]
    G = P // 16
    inv = 1.0 / (16.0 * h.shape[1])
    gt_r = lax.broadcasted_iota(jnp.int32, (G, P), 0)
    gt_c = lax.broadcasted_iota(jnp.int32, (G, P), 1)
    Gt = (gt_c // 16 == gt_r).astype(jnp.float32)
    gm_r = lax.broadcasted_iota(jnp.int32, (P, G), 0)
    gm_c = lax.broadcasted_iota(jnp.int32, (P, G), 1)
    Gm = (gm_r // 16 == gm_c).astype(jnp.float32)
    s1 = jnp.sum(jnp.dot(Gt, h, preferred_element_type=jnp.float32),
                 axis=1, keepdims=True)
    s2 = jnp.sum(jnp.dot(Gt, h * h, preferred_element_type=jnp.float32),
                 axis=1, keepdims=True)
    m = s1 * inv
    v = s2 * inv - m * m
    sc = lax.rsqrt(v + 1e-5)
    mr = jnp.dot(Gm, m, preferred_element_type=jnp.float32)
    sr = jnp.dot(Gm, sc, preferred_element_type=jnp.float32)
    hn = (h - mr) * sr
    return jnp.maximum(hn * gamma_rows + beta_rows, 0.0)


def _fusion_body(a2_ref, m2_ref, wf2_ref, bf2_ref, f0_ref, f1_ref,
                 w1a_ref, w1b_ref, w1c_ref, b1_ref, gam1_ref, bet1_ref,
                 w2_ref, b2_ref, gam2_ref, bet2_ref, pe_ref, o_ref):
    h2 = jnp.maximum(a2_ref[...] + m2_ref[...], 0.0)
    f2 = jnp.dot(h2, wf2_ref[...],
                 preferred_element_type=jnp.float32) + bf2_ref[...]
    h = (jnp.dot(f0_ref[...], w1a_ref[...], preferred_element_type=jnp.float32)
         + jnp.dot(f1_ref[...], w1b_ref[...], preferred_element_type=jnp.float32)
         + jnp.dot(f2, w1c_ref[...], preferred_element_type=jnp.float32)
         + b1_ref[...])
    h = _gn_relu(h, gam1_ref[...], bet1_ref[...])
    h = jnp.dot(h, w2_ref[...], preferred_element_type=jnp.float32) + b2_ref[...]
    h = _gn_relu(h, gam2_ref[...], bet2_ref[...])
    o_ref[...] = h + pe_ref[...]


def _segmax(stages, pbs=None):
    P = stages[0][1].shape[0]
    NW = 32
    RP = P // NW
    ns = len(stages)

    def make_plan(cap):
        plan = []
        words = 0
        for i, (idxf, g, k) in enumerate(stages):
            k = int(k)
            C = int(g.shape[1])
            if pbs is not None:
                PB = pbs[i]
            else:
                PB = max(1, min(128 // k, cap // (k * C), RP))
            plan.append((k, C, PB))
            words += RP * k + 2 * PB * k * C + RP * C
        return plan, words

    plan, words = make_plan(16384)
    if pbs is None and words > 126000:
        plan, words = make_plan(8192)
    mesh = plsc.VectorSubcoreMesh(core_axis_name="c", subcore_axis_name="s")

    scratch = []
    for (k, C, PB) in plan:
        scratch += [
            pltpu.VMEM((RP * k,), jnp.int32),
            pltpu.VMEM((PB * k, C), jnp.float32),
            pltpu.VMEM((PB * k, C), jnp.float32),
            pltpu.VMEM((RP, C), jnp.float32),
        ]
    scratch.append(pltpu.SemaphoreType.DMA)
    scratch.append(pltpu.SemaphoreType.DMA)

    out_type = [jax.ShapeDtypeStruct((P, C), jnp.float32) for (k, C, PB) in plan]

    def body(*refs):
        outs = refs[2 * ns:3 * ns]
        sem = refs[-2]
        osem = refs[-1]
        wid = lax.axis_index("s") * 2 + lax.axis_index("c")
        base = pl.multiple_of(wid * RP, RP)
        for si in range(ns):
            k, C, PB = plan[si]
            idx_v = refs[3 * ns + 4 * si]
            pltpu.sync_copy(refs[2 * si].at[pl.ds(base * k, RP * k)], idx_v)
        for si in range(ns):
            k, C, PB = plan[si]
            g_hbm = refs[2 * si + 1]
            o_hbm = outs[si]
            idx_v, rows0, rows1, out_v = refs[3 * ns + 4 * si:3 * ns + 4 * si + 4]
            W = PB * k
            NG = RP // PB
            nch = C // 16
            CH = min(nch, 16)
            ngrp = nch // CH

            def start(gi, buf, idx_v=idx_v, g_hbm=g_hbm, W=W, sem=sem):
                off = pl.multiple_of(gi * W, 8)
                pltpu.async_copy(g_hbm.at[idx_v.at[pl.ds(off, W)]], buf, sem)

            def drain(buf, g_hbm=g_hbm, W=W, sem=sem):
                pltpu.make_async_copy(g_hbm.at[pl.ds(0, W)], buf, sem).wait()

            def reduce(gi, buf, out_v=out_v, k=k, PB=PB, CH=CH, ngrp=ngrp):
                for pi in range(PB):
                    r0 = pi * k
                    for grp in range(ngrp):
                        off0 = grp * CH * 16

                        def red(blk, accs, buf=buf, r0=r0, off0=off0, CH=CH):
                            j = 1 + 3 * blk
                            for u in range(3):
                                accs = tuple(
                                    jnp.maximum(
                                        accs[t],
                                        buf[r0 + j + u, pl.ds(off0 + 16 * t, 16)])
                                    for t in range(CH))
                            return accs

                        accs = tuple(buf[r0, pl.ds(off0 + 16 * t, 16)]
                                     for t in range(CH))
                        accs = lax.fori_loop(0, (k - 1) // 3, red, accs)
                        for r in range(1 + 3 * ((k - 1) // 3), k):
                            accs = tuple(
                                jnp.maximum(accs[t],
                                            buf[r0 + r, pl.ds(off0 + 16 * t, 16)])
                                for t in range(CH))
                        prow = gi * PB + pi
                        for t in range(CH):
                            out_v[prow, pl.ds(off0 + 16 * t, 16)] = accs[t]

            start(0, rows0)

            def pair(pq, _, rows0=rows0, rows1=rows1, NG=NG,
                     start=start, drain=drain, reduce=reduce):
                g0 = 2 * pq
                drain(rows0)
                start(g0 + 1, rows1)
                reduce(g0, rows0)
                drain(rows1)

                @pl.when(pq < NG // 2 - 1)
                def _():
                    start(g0 + 2, rows0)

                reduce(g0 + 1, rows1)
                return 0

            lax.fori_loop(0, NG // 2, pair, 0)
            pltpu.async_copy(out_v, o_hbm.at[pl.ds(base, RP)], osem)
        for si in range(ns):
            k, C, PB = plan[si]
            out_v = refs[3 * ns + 4 * si + 3]
            pltpu.make_async_copy(outs[si].at[pl.ds(base, RP)], out_v, osem).wait()

    flat = []
    for (idxf, g, k) in stages:
        flat += [idxf, g]
    res = pl.kernel(body, out_type=out_type, mesh=mesh,
                    scratch_types=scratch)(*flat)
    if ns == 1:
        return [res[0] if isinstance(res, (tuple, list)) else res]
    return list(res)


def _pos_encoding(N, d):
    position = jnp.arange(N, dtype=jnp.float32)[:, None]
    div = jnp.exp(jnp.arange(0, d, 2, dtype=jnp.float32)
                  * (-math.log(10000.0) / d))
    pe = jnp.zeros((N, d), jnp.float32)
    pe = pe.at[:, 0::2].set(jnp.sin(position * div))
    pe = pe.at[:, 1::2].set(jnp.cos(position * div))
    return pe


def _split_w(W):
    C = W.shape[0] // 2
    return W[:C] - W[C:], W[C:]


def kernel(x, params):
    B, N, F = x.shape
    P = B * N

    xt = jnp.transpose(x, (0, 2, 1))
    negd = _neg_dist(x, xt)
    _, idx = lax.top_k(negd, 64)
    idx_flat = (idx.astype(jnp.int32)
                + (jnp.arange(B, dtype=jnp.int32) * N)[:, None, None]
                ).reshape(P, 64)
    idxf = {k: idx_flat[:, :k].reshape(-1) for k in (16, 32, 64)}

    xf = x.reshape(P, F)
    xp = jnp.pad(xf, ((0, 0), (0, 2)))

    enc = [params["enc0"], params["enc1"], params["enc2"]]
    ks = (16, 32, 64)

    entries = []
    for e in enc:
        W, b = e["layers"][0]
        wa, wg = _split_w(W)
        wa = jnp.pad(wa, ((0, 2), (0, 0)))
        wg = jnp.pad(wg, ((0, 2), (0, 0)))
        entries.append(("x_ag", xp, wa, wg, b[None, :]))
    a0, g0, a1, g1, a2, g2 = _tc_round(entries)
    m0, m1, m2 = _segmax([(idxf[16], g0, 16), (idxf[32], g1, 32), (idxf[64], g2, 64)])

    entries = []
    for e, a, m in zip(enc, (a0, a1, a2), (m0, m1, m2)):
        W, b = e["layers"][1]
        wa, wg = _split_w(W)
        entries.append(("am_ag", a, m, wa, wg, b[None, :]))
    a0, g0, a1, g1, a2, g2 = _tc_round(entries)
    (m1,) = _segmax([(idxf[32], g1, 32)], pbs=[2])
    m0, m2 = _segmax([(idxf[16], g0, 16), (idxf[64], g2, 64)], pbs=[4, 2])

    W2c, b2c = enc[2]["layers"][2]
    wa2, wg2 = _split_w(W2c)
    f0, f1, a2, g2 = _tc_round([
        ("am_f", a0, m0, enc[0]["final"][0], enc[0]["final"][1][None, :]),
        ("am_f", a1, m1, enc[1]["final"][0], enc[1]["final"][1][None, :]),
        ("am_ag", a2, m2, wa2, wg2, b2c[None, :]),
    ])
    (m2,) = _segmax([(idxf[64], g2, 64)])

    f = params["fusion"]
    W1 = f["W1"]
    pe = _pos_encoding(N, 512)
    pe2 = jnp.concatenate([pe, pe], axis=0)
    gam1 = jnp.concatenate([f["g1"], f["g1"]])[:, None]
    bet1 = jnp.concatenate([f["be1"], f["be1"]])[:, None]
    gam2 = jnp.concatenate([f["g2"], f["g2"]])[:, None]
    bet2 = jnp.concatenate([f["be2"], f["be2"]])[:, None]

    out = pl.pallas_call(
        _fusion_body,
        out_shape=jax.ShapeDtypeStruct((P, 512), jnp.float32),
    )(a2, m2, enc[2]["final"][0], enc[2]["final"][1][None, :], f0, f1,
      W1[0:256], W1[256:768], W1[768:1280], f["b1"][None, :], gam1, bet1,
      f["W2"], f["b2"][None, :], gam2, bet2, pe2)

    return out.reshape(B, N, 512)
